# baseline (speedup 1.0000x reference)
import functools

import jax
import jax.numpy as jnp
from jax import lax
from jax.experimental import pallas as pl
from jax.experimental.pallas import tpu as pltpu
from jax.experimental.pallas import tpu_sc as plsc

R = 128
N = 32768
L = 16
NC = 2
NS = 16
NW = NC * NS
ROWS_PER_W = R // NW
THRESH = 0.05


SEG = 512
SEG_CHUNKS = SEG // L
NSEG = N // SEG


def _permute(x, idx):
    return lax.gather(
        x, idx[:, None],
        lax.GatherDimensionNumbers(
            offset_dims=(), collapsed_slice_dims=(0,), start_index_map=(0,)
        ),
        slice_sizes=(1,),
        mode=lax.GatherScatterMode.PROMISE_IN_BOUNDS,
    )


def _bmax(x, lane):
    for s in (1, 2, 4, 8):
        x = jnp.maximum(x, _permute(x, lane ^ s))
    return x


def _process_row(buf_ref, seg_ref, mbuf, ibuf):
    lane = lax.iota(jnp.int32, L)
    neg = jnp.full((L,), -jnp.inf, jnp.float32)
    iz = jnp.zeros((L,), jnp.int32)
    neg1 = jnp.full((L,), -1, jnp.int32)

    def seg_body(sg, gm):
        base = sg * SEG
        accs = [buf_ref[pl.ds(base + a * L, L)] for a in range(4)]
        for c in range(4, SEG_CHUNKS):
            accs[c % 4] = jnp.maximum(accs[c % 4], buf_ref[pl.ds(base + c * L, L)])
        sm = jnp.maximum(jnp.maximum(accs[0], accs[1]),
                         jnp.maximum(accs[2], accs[3]))
        seg_ref[pl.ds(sg * L, L)] = sm
        return jnp.maximum(gm, sm)

    gm = lax.fori_loop(0, NSEG, seg_body, neg)

    t = gm
    for _ in range(2):
        tv = _bmax(t, lane)
        la = _bmax(jnp.where(t == tv, lane, neg1), lane)
        t = jnp.where(lane == la, neg, t)
    t3 = _bmax(t, lane)

    izero = jnp.zeros((L,), jnp.int32)
    acc_lo = izero
    acc_hi = izero
    for sg in range(NSEG):
        sm = seg_ref[pl.ds(sg * L, L)]
        b = 1 << (sg % 32)
        if b >= 1 << 31:
            b -= 1 << 32
        bit = jnp.where(sm >= t3, jnp.int32(b), 0)
        if sg < 32:
            acc_lo = acc_lo | bit
        else:
            acc_hi = acc_hi | bit

    def _bor(x):
        for s in (1, 2, 4, 8):
            x = x | _permute(x, lane ^ s)
        return x

    w_lo = _bor(acc_lo)[0]
    w_hi = _bor(acc_hi)[0]

    for t in range(3):
        mbuf[pl.ds(t * L, L)] = neg
        ibuf[pl.ds(t * L, L)] = iz

    def scan_body(sg, carry):
        w = jnp.where(sg < 32, w_lo, w_hi)
        bit = lax.shift_right_logical(w, sg & 31) & 1

        @pl.when(bit != 0)
        def _hit():
            base = sg * SEG
            acc = izero
            for u in range(SEG_CHUNKS):
                v = buf_ref[pl.ds(base + u * L, L)]
                b = 1 << u
                if b >= 1 << 31:
                    b -= 1 << 32
                acc = acc | jnp.where(v >= t3, jnp.int32(b), 0)
            w = _bor(acc)[0]
            for u in range(SEG_CHUNKS):
                b = 1 << u
                if b >= 1 << 31:
                    b -= 1 << 32
                cb = w & b

                @pl.when(cb != 0)
                def _chunk(u=u):
                    m1 = mbuf[pl.ds(0, L)]
                    m2 = mbuf[pl.ds(L, L)]
                    m3 = mbuf[pl.ds(2 * L, L)]
                    i1 = ibuf[pl.ds(0, L)]
                    i2 = ibuf[pl.ds(L, L)]
                    i3 = ibuf[pl.ds(2 * L, L)]
                    v = buf_ref[pl.ds(base + u * L, L)]
                    idx = lane + (base + u * L)
                    c1 = v >= m1
                    c2 = v >= m2
                    c3 = v >= m3
                    m3 = jnp.where(c3, jnp.where(c2, m2, v), m3)
                    i3 = jnp.where(c3, jnp.where(c2, i2, idx), i3)
                    m2 = jnp.where(c2, jnp.where(c1, m1, v), m2)
                    i2 = jnp.where(c2, jnp.where(c1, i1, idx), i2)
                    m1 = jnp.where(c1, v, m1)
                    i1 = jnp.where(c1, idx, i1)
                    mbuf[pl.ds(0, L)] = m1
                    mbuf[pl.ds(L, L)] = m2
                    mbuf[pl.ds(2 * L, L)] = m3
                    ibuf[pl.ds(0, L)] = i1
                    ibuf[pl.ds(L, L)] = i2
                    ibuf[pl.ds(2 * L, L)] = i3

        return carry

    lax.fori_loop(0, NSEG, scan_body, 0)
    m1 = mbuf[pl.ds(0, L)]
    m2 = mbuf[pl.ds(L, L)]
    m3 = mbuf[pl.ds(2 * L, L)]
    i1 = ibuf[pl.ds(0, L)]
    i2 = ibuf[pl.ds(L, L)]
    i3 = ibuf[pl.ds(2 * L, L)]

    res_v = jnp.zeros((L,), jnp.float32)
    res_i = jnp.zeros((L,), jnp.int32)
    for j in range(3):
        mv = _bmax(m1, lane)
        mi = _bmax(jnp.where(m1 == mv, i1, neg1), lane)
        sel = (m1 == mv) & (i1 == mi)
        m1 = jnp.where(sel, m2, m1)
        i1 = jnp.where(sel, i2, i1)
        m2 = jnp.where(sel, m3, m2)
        i2 = jnp.where(sel, i3, i2)
        res_v = jnp.where(lane == j, mv, res_v)
        res_i = jnp.where(lane == j, mi, res_i)
    return res_v, res_i


def _topk_sc(scores):
    mesh = plsc.VectorSubcoreMesh(core_axis_name="c", subcore_axis_name="s")

    @functools.partial(
        pl.kernel,
        out_type=jax.ShapeDtypeStruct((R, N), jnp.float32),
        mesh=mesh,
        scratch_types=[
            pltpu.VMEM((2, N), jnp.float32),
            pltpu.VMEM((N,), jnp.float32),
            pltpu.VMEM((NSEG * L,), jnp.float32),
            pltpu.VMEM((3 * L,), jnp.float32),
            pltpu.VMEM((3 * L,), jnp.int32),
            pltpu.VMEM((3, L), jnp.float32),
            pltpu.SemaphoreType.DMA,
            pltpu.SemaphoreType.DMA,
            pltpu.SemaphoreType.DMA,
            pltpu.SemaphoreType.DMA,
        ],
    )
    def topk(scores_hbm, out_hbm, rowbuf, outbuf, segbuf, mbuf, ibuf,
             patchbuf, sem0, sem1, osem, psem):
        wid = lax.axis_index("s") * NC + lax.axis_index("c")
        r0 = wid * ROWS_PER_W
        lane = lax.iota(jnp.int32, L)
        zvec = jnp.zeros((L,), jnp.float32)

        def zbody(i, c):
            for u in range(16):
                outbuf[pl.ds(i * (16 * L) + u * L, L)] = zvec
            return c

        lax.fori_loop(0, N // (16 * L), zbody, 0)

        def send_winners(res_v, res_i, row):
            res_vt = jnp.where(res_v > THRESH, res_v, 0.0)
            chs, lns, vls = [], [], []
            for j in range(3):
                ij = res_i[j]
                ch = (ij // L) * L
                chs.append(ch)
                lns.append(ij - ch)
                vls.append(res_vt[j])
            patches = []
            for j in range(3):
                chunk = jnp.zeros((L,), jnp.float32)
                for k in range(3):
                    hit = (lane + (chs[k] - chs[j])) == lns[k]
                    chunk = jnp.where(hit, vls[k], chunk)
                patchbuf[j] = chunk
                patches.append(
                    pltpu.async_copy(
                        patchbuf.at[j],
                        out_hbm.at[row].at[pl.ds(chs[j], L)],
                        psem,
                    )
                )
            return patches

        sems = [sem0, sem1]
        cps = [None, None]
        cps[0] = pltpu.async_copy(scores_hbm.at[r0], rowbuf.at[0], sems[0])
        zcp = pltpu.async_copy(outbuf, out_hbm.at[r0], osem)
        patches = []
        for rr in range(ROWS_PER_W):
            b = rr % 2
            if rr + 1 < ROWS_PER_W:
                nb = (rr + 1) % 2
                cps[nb] = pltpu.async_copy(
                    scores_hbm.at[r0 + rr + 1], rowbuf.at[nb], sems[nb]
                )
            cps[b].wait()
            res_v, res_i = _process_row(rowbuf.at[b], segbuf, mbuf, ibuf)
            for p in patches:
                p.wait()
            zcp.wait()
            patches = send_winners(res_v, res_i, r0 + rr)
            if rr + 1 < ROWS_PER_W:
                zcp = pltpu.async_copy(outbuf, out_hbm.at[r0 + rr + 1], osem)
        for p in patches:
            p.wait()

    return topk(scores)


def kernel(scores, k):
    del k
    return _topk_sc(scores)


# speedup vs baseline: 1.1379x; 1.1379x over previous
import functools

import jax
import jax.numpy as jnp
from jax import lax
from jax.experimental import pallas as pl
from jax.experimental.pallas import tpu as pltpu
from jax.experimental.pallas import tpu_sc as plsc

R = 128
N = 32768
L = 16
NC = 2
NS = 16
NW = NC * NS
ROWS_PER_W = R // NW
THRESH = 0.05


SEG = 512
SEG_CHUNKS = SEG // L
NSEG = N // SEG


def _permute(x, idx):
    return lax.gather(
        x, idx[:, None],
        lax.GatherDimensionNumbers(
            offset_dims=(), collapsed_slice_dims=(0,), start_index_map=(0,)
        ),
        slice_sizes=(1,),
        mode=lax.GatherScatterMode.PROMISE_IN_BOUNDS,
    )


def _bmax(x, lane):
    for s in (1, 2, 4, 8):
        x = jnp.maximum(x, _permute(x, lane ^ s))
    return x


def _process_row(buf_ref, seg_ref, mbuf, ibuf):
    lane = lax.iota(jnp.int32, L)
    neg = jnp.full((L,), -jnp.inf, jnp.float32)
    iz = jnp.zeros((L,), jnp.int32)
    neg1 = jnp.full((L,), -1, jnp.int32)

    def seg_body(sg, gm):
        base = sg * SEG
        accs = [buf_ref[pl.ds(base + a * L, L)] for a in range(4)]
        for c in range(4, SEG_CHUNKS):
            accs[c % 4] = jnp.maximum(accs[c % 4], buf_ref[pl.ds(base + c * L, L)])
        sm = jnp.maximum(jnp.maximum(accs[0], accs[1]),
                         jnp.maximum(accs[2], accs[3]))
        seg_ref[pl.ds(sg * L, L)] = sm
        return jnp.maximum(gm, sm)

    gm = lax.fori_loop(0, NSEG, seg_body, neg)

    t = gm
    for _ in range(2):
        tv = _bmax(t, lane)
        la = _bmax(jnp.where(t == tv, lane, neg1), lane)
        t = jnp.where(lane == la, neg, t)
    t3 = _bmax(t, lane)

    izero = jnp.zeros((L,), jnp.int32)
    acc_lo = izero
    acc_hi = izero
    for sg in range(NSEG):
        sm = seg_ref[pl.ds(sg * L, L)]
        b = 1 << (sg % 32)
        if b >= 1 << 31:
            b -= 1 << 32
        bit = jnp.where(sm >= t3, jnp.int32(b), 0)
        if sg < 32:
            acc_lo = acc_lo | bit
        else:
            acc_hi = acc_hi | bit

    def _bor(x):
        for s in (1, 2, 4, 8):
            x = x | _permute(x, lane ^ s)
        return x

    w_lo = _bor(acc_lo)[0]
    w_hi = _bor(acc_hi)[0]

    for t in range(3):
        mbuf[pl.ds(t * L, L)] = neg
        ibuf[pl.ds(t * L, L)] = iz

    def _insert_segment(sg):
        m1 = mbuf[pl.ds(0, L)]
        m2 = mbuf[pl.ds(L, L)]
        m3 = mbuf[pl.ds(2 * L, L)]
        i1 = ibuf[pl.ds(0, L)]
        i2 = ibuf[pl.ds(L, L)]
        i3 = ibuf[pl.ds(2 * L, L)]
        base = sg * SEG
        for u in range(SEG_CHUNKS):
            v = buf_ref[pl.ds(base + u * L, L)]
            idx = lane + (base + u * L)
            c1 = v >= m1
            c2 = v >= m2
            c3 = v >= m3
            m3 = jnp.where(c3, jnp.where(c2, m2, v), m3)
            i3 = jnp.where(c3, jnp.where(c2, i2, idx), i3)
            m2 = jnp.where(c2, jnp.where(c1, m1, v), m2)
            i2 = jnp.where(c2, jnp.where(c1, i1, idx), i2)
            m1 = jnp.where(c1, v, m1)
            i1 = jnp.where(c1, idx, i1)
        mbuf[pl.ds(0, L)] = m1
        mbuf[pl.ds(L, L)] = m2
        mbuf[pl.ds(2 * L, L)] = m3
        ibuf[pl.ds(0, L)] = i1
        ibuf[pl.ds(L, L)] = i2
        ibuf[pl.ds(2 * L, L)] = i3

    def scan_body(sg, carry):
        w = jnp.where(sg < 32, w_lo, w_hi)
        bit = lax.shift_right_logical(w, sg & 31) & 1

        @pl.when(bit != 0)
        def _hit():
            _insert_segment(sg)

        return carry

    lax.fori_loop(0, NSEG, scan_body, 0)
    m1 = mbuf[pl.ds(0, L)]
    m2 = mbuf[pl.ds(L, L)]
    m3 = mbuf[pl.ds(2 * L, L)]
    i1 = ibuf[pl.ds(0, L)]
    i2 = ibuf[pl.ds(L, L)]
    i3 = ibuf[pl.ds(2 * L, L)]

    res_v = jnp.zeros((L,), jnp.float32)
    res_i = jnp.zeros((L,), jnp.int32)
    for j in range(3):
        mv = _bmax(m1, lane)
        mi = _bmax(jnp.where(m1 == mv, i1, neg1), lane)
        sel = (m1 == mv) & (i1 == mi)
        m1 = jnp.where(sel, m2, m1)
        i1 = jnp.where(sel, i2, i1)
        m2 = jnp.where(sel, m3, m2)
        i2 = jnp.where(sel, i3, i2)
        res_v = jnp.where(lane == j, mv, res_v)
        res_i = jnp.where(lane == j, mi, res_i)
    return res_v, res_i


def _topk_sc(scores):
    mesh = plsc.VectorSubcoreMesh(core_axis_name="c", subcore_axis_name="s")

    @functools.partial(
        pl.kernel,
        out_type=jax.ShapeDtypeStruct((R, N), jnp.float32),
        mesh=mesh,
        scratch_types=[
            pltpu.VMEM((2, N), jnp.float32),
            pltpu.VMEM((N,), jnp.float32),
            pltpu.VMEM((NSEG * L,), jnp.float32),
            pltpu.VMEM((3 * L,), jnp.float32),
            pltpu.VMEM((3 * L,), jnp.int32),
            pltpu.VMEM((3, L), jnp.float32),
            pltpu.SemaphoreType.DMA,
            pltpu.SemaphoreType.DMA,
            pltpu.SemaphoreType.DMA,
            pltpu.SemaphoreType.DMA,
        ],
    )
    def topk(scores_hbm, out_hbm, rowbuf, outbuf, segbuf, mbuf, ibuf,
             patchbuf, sem0, sem1, osem, psem):
        wid = lax.axis_index("s") * NC + lax.axis_index("c")
        r0 = wid * ROWS_PER_W
        lane = lax.iota(jnp.int32, L)
        zvec = jnp.zeros((L,), jnp.float32)

        def zbody(i, c):
            for u in range(16):
                outbuf[pl.ds(i * (16 * L) + u * L, L)] = zvec
            return c

        lax.fori_loop(0, N // (16 * L), zbody, 0)

        def send_winners(res_v, res_i, row):
            res_vt = jnp.where(res_v > THRESH, res_v, 0.0)
            chs, lns, vls = [], [], []
            for j in range(3):
                ij = res_i[j]
                ch = (ij // L) * L
                chs.append(ch)
                lns.append(ij - ch)
                vls.append(res_vt[j])
            patches = []
            for j in range(3):
                chunk = jnp.zeros((L,), jnp.float32)
                for k in range(3):
                    hit = (lane + (chs[k] - chs[j])) == lns[k]
                    chunk = jnp.where(hit, vls[k], chunk)
                patchbuf[j] = chunk
                patches.append(
                    pltpu.async_copy(
                        patchbuf.at[j],
                        out_hbm.at[row].at[pl.ds(chs[j], L)],
                        psem,
                    )
                )
            return patches

        sems = [sem0, sem1]
        cps = [None, None]
        cps[0] = pltpu.async_copy(scores_hbm.at[r0], rowbuf.at[0], sems[0])
        zcp = pltpu.async_copy(outbuf, out_hbm.at[r0], osem)
        patches = []
        for rr in range(ROWS_PER_W):
            b = rr % 2
            if rr + 1 < ROWS_PER_W:
                nb = (rr + 1) % 2
                cps[nb] = pltpu.async_copy(
                    scores_hbm.at[r0 + rr + 1], rowbuf.at[nb], sems[nb]
                )
            cps[b].wait()
            res_v, res_i = _process_row(rowbuf.at[b], segbuf, mbuf, ibuf)
            for p in patches:
                p.wait()
            zcp.wait()
            patches = send_winners(res_v, res_i, r0 + rr)
            if rr + 1 < ROWS_PER_W:
                zcp = pltpu.async_copy(outbuf, out_hbm.at[r0 + rr + 1], osem)
        for p in patches:
            p.wait()

    return topk(scores)


def kernel(scores, k):
    del k
    return _topk_sc(scores)


# speedup vs baseline: 1.2351x; 1.0854x over previous
import functools

import jax
import jax.numpy as jnp
from jax import lax
from jax.experimental import pallas as pl
from jax.experimental.pallas import tpu as pltpu
from jax.experimental.pallas import tpu_sc as plsc

R = 128
N = 32768
L = 16
NC = 2
NS = 16
NW = NC * NS
ROWS_PER_W = R // NW
THRESH = 0.05


SEG = 512
SEG_CHUNKS = SEG // L
NSEG = N // SEG


def _permute(x, idx):
    return lax.gather(
        x, idx[:, None],
        lax.GatherDimensionNumbers(
            offset_dims=(), collapsed_slice_dims=(0,), start_index_map=(0,)
        ),
        slice_sizes=(1,),
        mode=lax.GatherScatterMode.PROMISE_IN_BOUNDS,
    )


def _bmax(x, lane):
    for s in (1, 2, 4, 8):
        x = jnp.maximum(x, _permute(x, lane ^ s))
    return x


def _process_row(buf_ref, seg_ref, mbuf, ibuf):
    lane = lax.iota(jnp.int32, L)
    neg = jnp.full((L,), -jnp.inf, jnp.float32)
    iz = jnp.zeros((L,), jnp.int32)
    neg1 = jnp.full((L,), -1, jnp.int32)

    def seg_body(sg, gm):
        base = sg * SEG
        accs = [buf_ref[pl.ds(base + a * L, L)] for a in range(4)]
        for c in range(4, SEG_CHUNKS):
            accs[c % 4] = jnp.maximum(accs[c % 4], buf_ref[pl.ds(base + c * L, L)])
        sm = jnp.maximum(jnp.maximum(accs[0], accs[1]),
                         jnp.maximum(accs[2], accs[3]))
        seg_ref[pl.ds(sg * L, L)] = sm
        return jnp.maximum(gm, sm)

    gm = lax.fori_loop(0, NSEG, seg_body, neg)

    t = gm
    for _ in range(2):
        tv = _bmax(t, lane)
        la = _bmax(jnp.where(t == tv, lane, neg1), lane)
        t = jnp.where(lane == la, neg, t)
    t3 = _bmax(t, lane)

    izero = jnp.zeros((L,), jnp.int32)
    acc_lo = izero
    acc_hi = izero
    for sg in range(NSEG):
        sm = seg_ref[pl.ds(sg * L, L)]
        b = 1 << (sg % 32)
        if b >= 1 << 31:
            b -= 1 << 32
        bit = jnp.where(sm >= t3, jnp.int32(b), 0)
        if sg < 32:
            acc_lo = acc_lo | bit
        else:
            acc_hi = acc_hi | bit

    def _bor(x):
        for s in (1, 2, 4, 8):
            x = x | _permute(x, lane ^ s)
        return x

    w_lo = _bor(acc_lo)[0]
    w_hi = _bor(acc_hi)[0]

    for t in range(3):
        mbuf[pl.ds(t * L, L)] = neg
        ibuf[pl.ds(t * L, L)] = iz

    def _insert_segment(sg):
        m1 = mbuf[pl.ds(0, L)]
        m2 = mbuf[pl.ds(L, L)]
        m3 = mbuf[pl.ds(2 * L, L)]
        i1 = ibuf[pl.ds(0, L)]
        i2 = ibuf[pl.ds(L, L)]
        i3 = ibuf[pl.ds(2 * L, L)]
        base = sg * SEG
        for u in range(SEG_CHUNKS):
            v = buf_ref[pl.ds(base + u * L, L)]
            idx = lane + (base + u * L)
            c1 = v >= m1
            c2 = v >= m2
            c3 = v >= m3
            m3 = jnp.where(c3, jnp.where(c2, m2, v), m3)
            i3 = jnp.where(c3, jnp.where(c2, i2, idx), i3)
            m2 = jnp.where(c2, jnp.where(c1, m1, v), m2)
            i2 = jnp.where(c2, jnp.where(c1, i1, idx), i2)
            m1 = jnp.where(c1, v, m1)
            i1 = jnp.where(c1, idx, i1)
        mbuf[pl.ds(0, L)] = m1
        mbuf[pl.ds(L, L)] = m2
        mbuf[pl.ds(2 * L, L)] = m3
        ibuf[pl.ds(0, L)] = i1
        ibuf[pl.ds(L, L)] = i2
        ibuf[pl.ds(2 * L, L)] = i3

    def scan_body(sg, carry):
        w = jnp.where(sg < 32, w_lo, w_hi)
        bit = lax.shift_right_logical(w, sg & 31) & 1

        @pl.when(bit != 0)
        def _hit():
            _insert_segment(sg)

        return carry

    lax.fori_loop(0, NSEG, scan_body, 0)
    m1 = mbuf[pl.ds(0, L)]
    m2 = mbuf[pl.ds(L, L)]
    m3 = mbuf[pl.ds(2 * L, L)]
    i1 = ibuf[pl.ds(0, L)]
    i2 = ibuf[pl.ds(L, L)]
    i3 = ibuf[pl.ds(2 * L, L)]

    res_v = jnp.zeros((L,), jnp.float32)
    res_i = jnp.zeros((L,), jnp.int32)
    for j in range(3):
        mv = _bmax(m1, lane)
        mi = _bmax(jnp.where(m1 == mv, i1, neg1), lane)
        sel = (m1 == mv) & (i1 == mi)
        m1 = jnp.where(sel, m2, m1)
        i1 = jnp.where(sel, i2, i1)
        m2 = jnp.where(sel, m3, m2)
        i2 = jnp.where(sel, i3, i2)
        res_v = jnp.where(lane == j, mv, res_v)
        res_i = jnp.where(lane == j, mi, res_i)
    return res_v, res_i


def _topk_sc(scores):
    mesh = plsc.VectorSubcoreMesh(core_axis_name="c", subcore_axis_name="s")

    @functools.partial(
        pl.kernel,
        out_type=jax.ShapeDtypeStruct((R, N), jnp.float32),
        mesh=mesh,
        scratch_types=[
            pltpu.VMEM((2, N), jnp.float32),
            pltpu.VMEM((N,), jnp.float32),
            pltpu.VMEM((NSEG * L,), jnp.float32),
            pltpu.VMEM((3 * L,), jnp.float32),
            pltpu.VMEM((3 * L,), jnp.int32),
            pltpu.VMEM((3, L), jnp.float32),
            pltpu.SemaphoreType.DMA((2,)),
            pltpu.SemaphoreType.DMA,
            pltpu.SemaphoreType.DMA,
        ],
    )
    def topk(scores_hbm, out_hbm, rowbuf, outbuf, segbuf, mbuf, ibuf,
             patchbuf, rsem, osem, psem):
        wid = lax.axis_index("s") * NC + lax.axis_index("c")
        r0 = wid * ROWS_PER_W
        lane = lax.iota(jnp.int32, L)
        zvec = jnp.zeros((L,), jnp.float32)

        def zbody(i, c):
            for u in range(16):
                outbuf[pl.ds(i * (16 * L) + u * L, L)] = zvec
            return c

        lax.fori_loop(0, N // (16 * L), zbody, 0)

        def send_winners(res_v, res_i, row):
            res_vt = jnp.where(res_v > THRESH, res_v, 0.0)
            chs, lns, vls = [], [], []
            for j in range(3):
                ij = res_i[j]
                ch = (ij // L) * L
                chs.append(ch)
                lns.append(ij - ch)
                vls.append(res_vt[j])
            patches = []
            for j in range(3):
                chunk = jnp.zeros((L,), jnp.float32)
                for k in range(3):
                    hit = (lane + (chs[k] - chs[j])) == lns[k]
                    chunk = jnp.where(hit, vls[k], chunk)
                patchbuf[j] = chunk
                patches.append(
                    pltpu.async_copy(
                        patchbuf.at[j],
                        out_hbm.at[row].at[pl.ds(chs[j], L)],
                        psem,
                    )
                )
            return patches

        pltpu.async_copy(scores_hbm.at[r0], rowbuf.at[0], rsem.at[0])
        pltpu.async_copy(outbuf, out_hbm.at[r0], osem)

        def row_body(rr, carry):
            b = rr & 1
            nb = 1 - b
            row = r0 + rr

            @pl.when(rr + 1 < ROWS_PER_W)
            def _prefetch():
                pltpu.async_copy(
                    scores_hbm.at[row + 1], rowbuf.at[nb], rsem.at[nb]
                )

            pltpu.make_async_copy(
                scores_hbm.at[row], rowbuf.at[b], rsem.at[b]
            ).wait()
            res_v, res_i = _process_row(rowbuf.at[b], segbuf, mbuf, ibuf)

            @pl.when(rr > 0)
            def _drain_patches():
                for j in range(3):
                    pltpu.make_async_copy(
                        patchbuf.at[j], out_hbm.at[row].at[pl.ds(0, L)], psem
                    ).wait()

            pltpu.make_async_copy(outbuf, out_hbm.at[row], osem).wait()
            send_winners(res_v, res_i, row)

            @pl.when(rr + 1 < ROWS_PER_W)
            def _next_zero():
                pltpu.async_copy(outbuf, out_hbm.at[row + 1], osem)

            return carry

        lax.fori_loop(0, ROWS_PER_W, row_body, 0)
        for j in range(3):
            pltpu.make_async_copy(
                patchbuf.at[j], out_hbm.at[r0].at[pl.ds(0, L)], psem
            ).wait()

    return topk(scores)


def kernel(scores, k):
    del k
    return _topk_sc(scores)


# speedup vs baseline: 1.2640x; 1.0234x over previous
import functools

import jax
import jax.numpy as jnp
from jax import lax
from jax.experimental import pallas as pl
from jax.experimental.pallas import tpu as pltpu
from jax.experimental.pallas import tpu_sc as plsc

R = 128
N = 32768
L = 16
NC = 2
NS = 16
NW = NC * NS
ROWS_PER_W = R // NW
THRESH = 0.05


SEG = 512
SEG_CHUNKS = SEG // L
NSEG = N // SEG


def _permute(x, idx):
    return lax.gather(
        x, idx[:, None],
        lax.GatherDimensionNumbers(
            offset_dims=(), collapsed_slice_dims=(0,), start_index_map=(0,)
        ),
        slice_sizes=(1,),
        mode=lax.GatherScatterMode.PROMISE_IN_BOUNDS,
    )


def _bmax(x, lane):
    for s in (1, 2, 4, 8):
        x = jnp.maximum(x, _permute(x, lane ^ s))
    return x


def _process_row(buf_ref, seg_ref, mbuf, ibuf):
    lane = lax.iota(jnp.int32, L)
    neg = jnp.full((L,), -jnp.inf, jnp.float32)
    iz = jnp.zeros((L,), jnp.int32)
    neg1 = jnp.full((L,), -1, jnp.int32)

    def seg_body(sg, gm):
        base = sg * SEG
        accs = [buf_ref[pl.ds(base + a * L, L)] for a in range(4)]
        for c in range(4, SEG_CHUNKS):
            accs[c % 4] = jnp.maximum(accs[c % 4], buf_ref[pl.ds(base + c * L, L)])
        sm = jnp.maximum(jnp.maximum(accs[0], accs[1]),
                         jnp.maximum(accs[2], accs[3]))
        seg_ref[pl.ds(sg * L, L)] = sm
        return jnp.maximum(gm, sm)

    gm = lax.fori_loop(0, NSEG, seg_body, neg)

    t = gm
    for _ in range(2):
        tv = _bmax(t, lane)
        la = _bmax(jnp.where(t == tv, lane, neg1), lane)
        t = jnp.where(lane == la, neg, t)
    t3 = _bmax(t, lane)

    izero = jnp.zeros((L,), jnp.int32)
    acc_lo = izero
    acc_hi = izero
    for sg in range(NSEG):
        sm = seg_ref[pl.ds(sg * L, L)]
        b = 1 << (sg % 32)
        if b >= 1 << 31:
            b -= 1 << 32
        bit = jnp.where(sm >= t3, jnp.int32(b), 0)
        if sg < 32:
            acc_lo = acc_lo | bit
        else:
            acc_hi = acc_hi | bit

    def _bor(x):
        for s in (1, 2, 4, 8):
            x = x | _permute(x, lane ^ s)
        return x

    w_lo = _bor(acc_lo)[0]
    w_hi = _bor(acc_hi)[0]

    for t in range(3):
        mbuf[pl.ds(t * L, L)] = neg
        ibuf[pl.ds(t * L, L)] = iz

    def _insert_segment(sg):
        m1 = mbuf[pl.ds(0, L)]
        m2 = mbuf[pl.ds(L, L)]
        m3 = mbuf[pl.ds(2 * L, L)]
        i1 = ibuf[pl.ds(0, L)]
        i2 = ibuf[pl.ds(L, L)]
        i3 = ibuf[pl.ds(2 * L, L)]
        base = sg * SEG
        for u in range(SEG_CHUNKS):
            v = buf_ref[pl.ds(base + u * L, L)]
            idx = lane + (base + u * L)
            c1 = v >= m1
            c2 = v >= m2
            c3 = v >= m3
            m3 = jnp.where(c3, jnp.where(c2, m2, v), m3)
            i3 = jnp.where(c3, jnp.where(c2, i2, idx), i3)
            m2 = jnp.where(c2, jnp.where(c1, m1, v), m2)
            i2 = jnp.where(c2, jnp.where(c1, i1, idx), i2)
            m1 = jnp.where(c1, v, m1)
            i1 = jnp.where(c1, idx, i1)
        mbuf[pl.ds(0, L)] = m1
        mbuf[pl.ds(L, L)] = m2
        mbuf[pl.ds(2 * L, L)] = m3
        ibuf[pl.ds(0, L)] = i1
        ibuf[pl.ds(L, L)] = i2
        ibuf[pl.ds(2 * L, L)] = i3

    def scan_body(sg, carry):
        w = jnp.where(sg < 32, w_lo, w_hi)
        bit = lax.shift_right_logical(w, sg & 31) & 1

        @pl.when(bit != 0)
        def _hit():
            _insert_segment(sg)

        return carry

    lax.fori_loop(0, NSEG, scan_body, 0)
    m1 = mbuf[pl.ds(0, L)]
    m2 = mbuf[pl.ds(L, L)]
    m3 = mbuf[pl.ds(2 * L, L)]
    i1 = ibuf[pl.ds(0, L)]
    i2 = ibuf[pl.ds(L, L)]
    i3 = ibuf[pl.ds(2 * L, L)]

    res_v = jnp.zeros((L,), jnp.float32)
    res_i = jnp.zeros((L,), jnp.int32)
    for j in range(3):
        mv = _bmax(m1, lane)
        mi = _bmax(jnp.where(m1 == mv, i1, neg1), lane)
        sel = (m1 == mv) & (i1 == mi)
        m1 = jnp.where(sel, m2, m1)
        i1 = jnp.where(sel, i2, i1)
        m2 = jnp.where(sel, m3, m2)
        i2 = jnp.where(sel, i3, i2)
        res_v = jnp.where(lane == j, mv, res_v)
        res_i = jnp.where(lane == j, mi, res_i)
    return res_v, res_i


def _topk_sc(scores):
    mesh = plsc.VectorSubcoreMesh(core_axis_name="c", subcore_axis_name="s")

    @functools.partial(
        pl.kernel,
        out_type=jax.ShapeDtypeStruct((R, N), jnp.float32),
        mesh=mesh,
        scratch_types=[
            pltpu.VMEM((2, N), jnp.float32),
            pltpu.VMEM((N,), jnp.float32),
            pltpu.VMEM((NSEG * L,), jnp.float32),
            pltpu.VMEM((3 * L,), jnp.float32),
            pltpu.VMEM((3 * L,), jnp.int32),
            pltpu.VMEM((3, L), jnp.float32),
            pltpu.SemaphoreType.DMA((2,)),
            pltpu.SemaphoreType.DMA,
            pltpu.SemaphoreType.DMA,
        ],
    )
    def topk(scores_hbm, out_hbm, rowbuf, outbuf, segbuf, mbuf, ibuf,
             patchbuf, rsem, osem, psem):
        wid = lax.axis_index("s") * NC + lax.axis_index("c")
        r0 = wid * ROWS_PER_W
        lane = lax.iota(jnp.int32, L)
        zvec = jnp.zeros((L,), jnp.float32)

        pltpu.async_copy(scores_hbm.at[r0], rowbuf.at[0], rsem.at[0])

        def zbody(i, c):
            for u in range(16):
                outbuf[pl.ds(i * (16 * L) + u * L, L)] = zvec
            return c

        lax.fori_loop(0, N // (16 * L), zbody, 0)

        def send_winners(res_v, res_i, row):
            res_vt = jnp.where(res_v > THRESH, res_v, 0.0)
            chs, lns, vls = [], [], []
            for j in range(3):
                ij = res_i[j]
                ch = (ij // L) * L
                chs.append(ch)
                lns.append(ij - ch)
                vls.append(res_vt[j])
            patches = []
            for j in range(3):
                chunk = jnp.zeros((L,), jnp.float32)
                for k in range(3):
                    hit = (lane + (chs[k] - chs[j])) == lns[k]
                    chunk = jnp.where(hit, vls[k], chunk)
                patchbuf[j] = chunk
                patches.append(
                    pltpu.async_copy(
                        patchbuf.at[j],
                        out_hbm.at[row].at[pl.ds(chs[j], L)],
                        psem,
                    )
                )
            return patches

        pltpu.async_copy(outbuf, out_hbm.at[r0], osem)

        def row_body(rr, carry):
            b = rr & 1
            nb = 1 - b
            row = r0 + rr

            @pl.when(rr + 1 < ROWS_PER_W)
            def _prefetch():
                pltpu.async_copy(
                    scores_hbm.at[row + 1], rowbuf.at[nb], rsem.at[nb]
                )

            pltpu.make_async_copy(
                scores_hbm.at[row], rowbuf.at[b], rsem.at[b]
            ).wait()
            res_v, res_i = _process_row(rowbuf.at[b], segbuf, mbuf, ibuf)

            @pl.when(rr > 0)
            def _drain_patches():
                for j in range(3):
                    pltpu.make_async_copy(
                        patchbuf.at[j], out_hbm.at[row].at[pl.ds(0, L)], psem
                    ).wait()

            pltpu.make_async_copy(outbuf, out_hbm.at[row], osem).wait()
            send_winners(res_v, res_i, row)

            @pl.when(rr + 1 < ROWS_PER_W)
            def _next_zero():
                pltpu.async_copy(outbuf, out_hbm.at[row + 1], osem)

            return carry

        lax.fori_loop(0, ROWS_PER_W, row_body, 0)
        for j in range(3):
            pltpu.make_async_copy(
                patchbuf.at[j], out_hbm.at[r0].at[pl.ds(0, L)], psem
            ).wait()

    return topk(scores)


def kernel(scores, k):
    del k
    return _topk_sc(scores)


# speedup vs baseline: 1.2686x; 1.0037x over previous
import functools

import jax
import jax.numpy as jnp
from jax import lax
from jax.experimental import pallas as pl
from jax.experimental.pallas import tpu as pltpu
from jax.experimental.pallas import tpu_sc as plsc

R = 128
N = 32768
L = 16
NC = 2
NS = 16
NW = NC * NS
ROWS_PER_W = R // NW
THRESH = 0.05


SEG = 512
SEG_CHUNKS = SEG // L
NSEG = N // SEG


def _permute(x, idx):
    return lax.gather(
        x, idx[:, None],
        lax.GatherDimensionNumbers(
            offset_dims=(), collapsed_slice_dims=(0,), start_index_map=(0,)
        ),
        slice_sizes=(1,),
        mode=lax.GatherScatterMode.PROMISE_IN_BOUNDS,
    )


def _bmax(x, lane):
    for s in (1, 2, 4, 8):
        x = jnp.maximum(x, _permute(x, lane ^ s))
    return x


def _process_row(buf_ref, seg_ref, mbuf, ibuf):
    lane = lax.iota(jnp.int32, L)
    neg = jnp.full((L,), -jnp.inf, jnp.float32)
    iz = jnp.zeros((L,), jnp.int32)
    neg1 = jnp.full((L,), -1, jnp.int32)

    def seg_body(sg, gm):
        base = sg * SEG
        accs = [buf_ref[pl.ds(base + a * L, L)] for a in range(4)]
        for c in range(4, SEG_CHUNKS):
            accs[c % 4] = jnp.maximum(accs[c % 4], buf_ref[pl.ds(base + c * L, L)])
        sm = jnp.maximum(jnp.maximum(accs[0], accs[1]),
                         jnp.maximum(accs[2], accs[3]))
        seg_ref[pl.ds(sg * L, L)] = sm
        return jnp.maximum(gm, sm)

    gm = lax.fori_loop(0, NSEG, seg_body, neg)

    t = gm
    for _ in range(2):
        tv = _bmax(t, lane)
        la = _bmax(jnp.where(t == tv, lane, neg1), lane)
        t = jnp.where(lane == la, neg, t)
    t3 = _bmax(t, lane)

    izero = jnp.zeros((L,), jnp.int32)
    acc_lo = izero
    acc_hi = izero
    for sg in range(NSEG):
        sm = seg_ref[pl.ds(sg * L, L)]
        b = 1 << (sg % 32)
        if b >= 1 << 31:
            b -= 1 << 32
        bit = jnp.where(sm >= t3, jnp.int32(b), 0)
        if sg < 32:
            acc_lo = acc_lo | bit
        else:
            acc_hi = acc_hi | bit

    def _bor(x):
        for s in (1, 2, 4, 8):
            x = x | _permute(x, lane ^ s)
        return x

    w_lo = _bor(acc_lo)[0]
    w_hi = _bor(acc_hi)[0]

    for t in range(3):
        mbuf[pl.ds(t * L, L)] = neg
        ibuf[pl.ds(t * L, L)] = iz

    def _insert_segment(sg):
        m1 = mbuf[pl.ds(0, L)]
        m2 = mbuf[pl.ds(L, L)]
        m3 = mbuf[pl.ds(2 * L, L)]
        i1 = ibuf[pl.ds(0, L)]
        i2 = ibuf[pl.ds(L, L)]
        i3 = ibuf[pl.ds(2 * L, L)]
        base = sg * SEG
        for u in range(SEG_CHUNKS):
            v = buf_ref[pl.ds(base + u * L, L)]
            idx = lane + (base + u * L)
            c1 = v >= m1
            c2 = v >= m2
            c3 = v >= m3
            m3 = jnp.where(c3, jnp.where(c2, m2, v), m3)
            i3 = jnp.where(c3, jnp.where(c2, i2, idx), i3)
            m2 = jnp.where(c2, jnp.where(c1, m1, v), m2)
            i2 = jnp.where(c2, jnp.where(c1, i1, idx), i2)
            m1 = jnp.where(c1, v, m1)
            i1 = jnp.where(c1, idx, i1)
        mbuf[pl.ds(0, L)] = m1
        mbuf[pl.ds(L, L)] = m2
        mbuf[pl.ds(2 * L, L)] = m3
        ibuf[pl.ds(0, L)] = i1
        ibuf[pl.ds(L, L)] = i2
        ibuf[pl.ds(2 * L, L)] = i3

    def scan_body(i, carry):
        w = jnp.where(i < 16, w_lo, w_hi)
        for h in range(2):
            sg = i * 2 + h
            bit = lax.shift_right_logical(w, sg & 31) & 1

            @pl.when(bit != 0)
            def _hit(sg=sg):
                _insert_segment(sg)

        return carry

    lax.fori_loop(0, NSEG // 2, scan_body, 0)
    m1 = mbuf[pl.ds(0, L)]
    m2 = mbuf[pl.ds(L, L)]
    m3 = mbuf[pl.ds(2 * L, L)]
    i1 = ibuf[pl.ds(0, L)]
    i2 = ibuf[pl.ds(L, L)]
    i3 = ibuf[pl.ds(2 * L, L)]

    res_v = jnp.zeros((L,), jnp.float32)
    res_i = jnp.zeros((L,), jnp.int32)
    for j in range(3):
        mv = _bmax(m1, lane)
        mi = _bmax(jnp.where(m1 == mv, i1, neg1), lane)
        sel = (m1 == mv) & (i1 == mi)
        m1 = jnp.where(sel, m2, m1)
        i1 = jnp.where(sel, i2, i1)
        m2 = jnp.where(sel, m3, m2)
        i2 = jnp.where(sel, i3, i2)
        res_v = jnp.where(lane == j, mv, res_v)
        res_i = jnp.where(lane == j, mi, res_i)
    return res_v, res_i


def _topk_sc(scores):
    mesh = plsc.VectorSubcoreMesh(core_axis_name="c", subcore_axis_name="s")

    @functools.partial(
        pl.kernel,
        out_type=jax.ShapeDtypeStruct((R, N), jnp.float32),
        mesh=mesh,
        scratch_types=[
            pltpu.VMEM((2, N), jnp.float32),
            pltpu.VMEM((N,), jnp.float32),
            pltpu.VMEM((NSEG * L,), jnp.float32),
            pltpu.VMEM((3 * L,), jnp.float32),
            pltpu.VMEM((3 * L,), jnp.int32),
            pltpu.VMEM((3, L), jnp.float32),
            pltpu.SemaphoreType.DMA((2,)),
            pltpu.SemaphoreType.DMA,
            pltpu.SemaphoreType.DMA,
        ],
    )
    def topk(scores_hbm, out_hbm, rowbuf, outbuf, segbuf, mbuf, ibuf,
             patchbuf, rsem, osem, psem):
        wid = lax.axis_index("s") * NC + lax.axis_index("c")
        r0 = wid * ROWS_PER_W
        lane = lax.iota(jnp.int32, L)
        zvec = jnp.zeros((L,), jnp.float32)

        pltpu.async_copy(scores_hbm.at[r0], rowbuf.at[0], rsem.at[0])

        def zbody(i, c):
            for u in range(16):
                outbuf[pl.ds(i * (16 * L) + u * L, L)] = zvec
            return c

        lax.fori_loop(0, N // (16 * L), zbody, 0)

        def send_winners(res_v, res_i, row):
            res_vt = jnp.where(res_v > THRESH, res_v, 0.0)
            chs, lns, vls = [], [], []
            for j in range(3):
                ij = res_i[j]
                ch = (ij // L) * L
                chs.append(ch)
                lns.append(ij - ch)
                vls.append(res_vt[j])
            patches = []
            for j in range(3):
                chunk = jnp.zeros((L,), jnp.float32)
                for k in range(3):
                    hit = (lane + (chs[k] - chs[j])) == lns[k]
                    chunk = jnp.where(hit, vls[k], chunk)
                patchbuf[j] = chunk
                patches.append(
                    pltpu.async_copy(
                        patchbuf.at[j],
                        out_hbm.at[row].at[pl.ds(chs[j], L)],
                        psem,
                    )
                )
            return patches

        pltpu.async_copy(outbuf, out_hbm.at[r0], osem)

        def row_body(rr, carry):
            b = rr & 1
            nb = 1 - b
            row = r0 + rr

            @pl.when(rr + 1 < ROWS_PER_W)
            def _prefetch():
                pltpu.async_copy(
                    scores_hbm.at[row + 1], rowbuf.at[nb], rsem.at[nb]
                )

            pltpu.make_async_copy(
                scores_hbm.at[row], rowbuf.at[b], rsem.at[b]
            ).wait()
            res_v, res_i = _process_row(rowbuf.at[b], segbuf, mbuf, ibuf)

            @pl.when(rr > 0)
            def _drain_patches():
                for j in range(3):
                    pltpu.make_async_copy(
                        patchbuf.at[j], out_hbm.at[row].at[pl.ds(0, L)], psem
                    ).wait()

            pltpu.make_async_copy(outbuf, out_hbm.at[row], osem).wait()
            send_winners(res_v, res_i, row)

            @pl.when(rr + 1 < ROWS_PER_W)
            def _next_zero():
                pltpu.async_copy(outbuf, out_hbm.at[row + 1], osem)

            return carry

        lax.fori_loop(0, ROWS_PER_W, row_body, 0)
        for j in range(3):
            pltpu.make_async_copy(
                patchbuf.at[j], out_hbm.at[r0].at[pl.ds(0, L)], psem
            ).wait()

    return topk(scores)


def kernel(scores, k):
    del k
    return _topk_sc(scores)
